# baseline (speedup 1.0000x reference)
import functools

import jax
import jax.numpy as jnp
from jax import lax
from jax.experimental import pallas as pl
from jax.experimental.pallas import tpu as pltpu
from jax.experimental.pallas import tpu_sc as plsc

_N_GENES = 100000
_N_DH = 64
_N_LAT = 50
_N_GOI = 4096

_NC, _NS = 2, 16
_NW = _NC * _NS
_B_PER_W = _N_GOI // _NW

_KFLAT = _N_DH * _N_LAT
_GRID = 16
_BG = _N_GOI // _GRID
_NROW2 = _N_GENES // 2
_BR2 = 3128


def _sc_body(idx_hbm, slope_hbm, scale_hbm, shift_hbm,
             h_hbm, a_hbm, s_hbm,
             idx_v, rows_v, sem):
    wid = lax.axis_index("s") * _NC + lax.axis_index("c")
    base = wid * _B_PER_W
    pltpu.sync_copy(idx_hbm.at[pl.ds(base, _B_PER_W)], idx_v)

    def gather_xform(table_hbm, out_hbm, is_exp):
        def chunk_dma(c, carry):
            v = idx_v[pl.ds(c * 16, 16)]
            descs = []
            for lane in range(16):
                r = v[lane]
                descs.append(pltpu.async_copy(
                    table_hbm.at[pl.ds(r, 1), :],
                    rows_v.at[pl.ds(c * 16 + lane, 1), :], sem))
            for d in descs:
                d.wait()
            return carry

        lax.fori_loop(0, _B_PER_W // 16, chunk_dma, 0)

        def row(i, carry):
            for j in range(_N_DH // 16):
                sl = rows_v[i, pl.ds(j * 16, 16)]
                if is_exp:
                    sl = jnp.exp(sl)
                rows_v[i, pl.ds(j * 16, 16)] = sl * 0.5
            return carry

        lax.fori_loop(0, _B_PER_W, row, 0)
        pltpu.sync_copy(rows_v, out_hbm.at[pl.ds(base, _B_PER_W)])

    gather_xform(slope_hbm, h_hbm, False)
    gather_xform(scale_hbm, a_hbm, True)
    gather_xform(shift_hbm, s_hbm, False)


@functools.cache
def _make_sc_gather():
    return pl.kernel(
        _sc_body,
        out_type=(
            jax.ShapeDtypeStruct((_N_GOI, _N_DH), jnp.float32),
            jax.ShapeDtypeStruct((_N_GOI, _N_DH), jnp.float32),
            jax.ShapeDtypeStruct((_N_GOI, _N_DH), jnp.float32),
        ),
        mesh=plsc.VectorSubcoreMesh(core_axis_name="c", subcore_axis_name="s",
                                    num_cores=_NC, num_subcores=_NS),
        scratch_types=[
            pltpu.VMEM((_B_PER_W,), jnp.int32),
            pltpu.VMEM((_B_PER_W, _N_DH), jnp.float32),
            pltpu.SemaphoreType.DMA,
        ],
    )


def _tc_body(a_ref, s_ref, h_ref, el_ref, e_ref, w2_ref, elo_ref,
             dh_ref, do_ref):
    z = jnp.dot(a_ref[...], el_ref[...], preferred_element_type=jnp.float32)
    z = z + jnp.dot(s_ref[...], e_ref[...], preferred_element_type=jnp.float32)
    hs = jnp.dot(h_ref[...], e_ref[...], preferred_element_type=jnp.float32)
    dh_ref[...] = hs * (1.0 + jnp.tanh(z))
    do_ref[...] = jnp.dot(w2_ref[...], elo_ref[...],
                          preferred_element_type=jnp.float32)


@functools.partial(jax.jit, static_argnames=())
def _tc_expand(a, s, h, el, e64, w2, elo):
    return pl.pallas_call(
        _tc_body,
        grid=(_GRID,),
        in_specs=[
            pl.BlockSpec((_BG, _N_DH), lambda i: (i, 0)),
            pl.BlockSpec((_BG, _N_DH), lambda i: (i, 0)),
            pl.BlockSpec((_BG, _N_DH), lambda i: (i, 0)),
            pl.BlockSpec((_N_DH, _KFLAT), lambda i: (0, 0)),
            pl.BlockSpec((_N_DH, _KFLAT), lambda i: (0, 0)),
            pl.BlockSpec((_BR2, 2), lambda i: (i, 0)),
            pl.BlockSpec((2, 2 * _N_LAT), lambda i: (0, 0)),
        ],
        out_specs=[
            pl.BlockSpec((_BG, _KFLAT), lambda i: (i, 0)),
            pl.BlockSpec((_BR2, 2 * _N_LAT), lambda i: (i, 0)),
        ],
        out_shape=[
            jax.ShapeDtypeStruct((_N_GOI, _KFLAT), jnp.float32),
            jax.ShapeDtypeStruct((_NROW2, 2 * _N_LAT), jnp.float32),
        ],
    )(a, s, h, el, e64, w2, elo)


def kernel(latent, genes_oi, W_height_slope, W_height_scale, W_height_shift,
           W_overall_slope):
    latf = jnp.tile(latent, _N_DH)
    e64 = (lax.broadcasted_iota(jnp.int32, (_N_DH, _KFLAT), 1) // _N_LAT
           == lax.broadcasted_iota(jnp.int32, (_N_DH, _KFLAT), 0)
           ).astype(jnp.float32)
    el = e64 * latf[None, :]
    lat2 = jnp.concatenate([latent, latent])
    elo = (lax.broadcasted_iota(jnp.int32, (2, 2 * _N_LAT), 1) // _N_LAT
           == lax.broadcasted_iota(jnp.int32, (2, 2 * _N_LAT), 0)
           ).astype(jnp.float32) * lat2[None, :]

    h, a, s = _make_sc_gather()(genes_oi, W_height_slope, W_height_scale,
                                W_height_shift)

    w2 = W_overall_slope.reshape(_NROW2, 2)
    dh_flat, do_flat = _tc_expand(a, s, h, el, e64, w2, elo)

    delta_height = dh_flat.reshape(_N_GOI, _N_DH, _N_LAT)
    delta_overall = do_flat.reshape(_N_GENES, 1, _N_LAT)
    return (delta_height, delta_overall)


# speedup vs baseline: 2.2096x; 2.2096x over previous
import functools

import jax
import jax.numpy as jnp
from jax import lax
from jax.experimental import pallas as pl
from jax.experimental.pallas import tpu as pltpu
from jax.experimental.pallas import tpu_sc as plsc

_N_GENES = 100000
_N_DH = 64
_N_LAT = 50
_N_GOI = 4096
_HALF = _N_GENES // 2

_NC, _NS = 2, 16
_NW = _NC * _NS
_B_PER_W = _N_GOI // _NW

_GRID = 16
_BG = _N_GOI // _GRID

_OV_BLK = 8192
_OV_GRID = -(-_N_GENES // _OV_BLK)

_PK_BLK = 2048
_PK_GRID = -(-_HALF // _PK_BLK)


def _pk_body(lo_ref, hi_ref, out_ref):
    out_ref[...] = jnp.concatenate([lo_ref[...].T, hi_ref[...].T], axis=1)


@jax.jit
def _repack(table_t):
    return pl.pallas_call(
        _pk_body,
        grid=(_PK_GRID,),
        in_specs=[
            pl.BlockSpec((_N_DH, _PK_BLK), lambda i: (0, i)),
            pl.BlockSpec((_N_DH, _PK_BLK),
                         lambda i: (0, i + _HALF // _PK_BLK)),
        ],
        out_specs=pl.BlockSpec((_PK_BLK, 2 * _N_DH), lambda i: (i, 0)),
        out_shape=jax.ShapeDtypeStruct((_HALF, 2 * _N_DH), jnp.float32),
    )(table_t, table_t)


def _sc_body(idx_hbm, ptab_hbm, out_hbm, idx_v, rows_v, sem):
    wid = lax.axis_index("s") * _NC + lax.axis_index("c")
    base = wid * _B_PER_W
    pltpu.sync_copy(idx_hbm.at[pl.ds(base, _B_PER_W)], idx_v)

    def fix(c, carry):
        v = idx_v[pl.ds(c * 16, 16)]
        idx_v[pl.ds(c * 16, 16)] = v - jnp.where(v >= _HALF, _HALF, 0)
        return carry

    lax.fori_loop(0, _B_PER_W // 16, fix, 0)
    pltpu.async_copy(ptab_hbm.at[idx_v], rows_v, sem).wait()
    pltpu.sync_copy(rows_v, out_hbm.at[pl.ds(base, _B_PER_W)])


@functools.cache
def _make_sc_gather():
    return pl.kernel(
        _sc_body,
        out_type=jax.ShapeDtypeStruct((_N_GOI, 2 * _N_DH), jnp.float32),
        mesh=plsc.VectorSubcoreMesh(core_axis_name="c", subcore_axis_name="s",
                                    num_cores=_NC, num_subcores=_NS),
        scratch_types=[
            pltpu.VMEM((_B_PER_W,), jnp.int32),
            pltpu.VMEM((_B_PER_W, 2 * _N_DH), jnp.float32),
            pltpu.SemaphoreType.DMA,
        ],
    )


def _dh_body(sl_ref, sc_ref, sh_ref, hf_ref, lat_ref, out_ref):
    hf = hf_ref[...] > 0

    def sel(ref):
        x = ref[...]
        return jnp.where(hf, x[:, _N_DH:], x[:, : _N_DH])

    h3 = (0.5 * sel(sl_ref)).T[None, :, :]
    a3 = (0.5 * jnp.exp(sel(sc_ref))).T[None, :, :]
    s3 = (0.5 * sel(sh_ref)).T[None, :, :]
    lat3 = lat_ref[...][:, :, None]
    t = jnp.tanh(a3 * lat3 + s3)
    out_ref[...] = h3 * (1.0 + t)


@jax.jit
def _dh_expand(g_sl, g_sc, g_sh, hf, lat_col):
    blk = pl.BlockSpec((_BG, 2 * _N_DH), lambda i: (i, 0))
    return pl.pallas_call(
        _dh_body,
        grid=(_GRID,),
        in_specs=[
            blk, blk, blk,
            pl.BlockSpec((_BG, 1), lambda i: (i, 0)),
            pl.BlockSpec((_N_LAT, 1), lambda i: (0, 0)),
        ],
        out_specs=pl.BlockSpec((_N_LAT, _N_DH, _BG), lambda i: (0, 0, i)),
        out_shape=jax.ShapeDtypeStruct((_N_LAT, _N_DH, _N_GOI), jnp.float32),
    )(g_sl, g_sc, g_sh, hf, lat_col)


def _ov_body(w_ref, lat_ref, out_ref):
    out_ref[...] = lat_ref[...] * w_ref[...]


@jax.jit
def _ov_expand(w_row, lat_col):
    return pl.pallas_call(
        _ov_body,
        grid=(_OV_GRID,),
        in_specs=[
            pl.BlockSpec((1, _OV_BLK), lambda i: (0, i)),
            pl.BlockSpec((_N_LAT, 1), lambda i: (0, 0)),
        ],
        out_specs=pl.BlockSpec((_N_LAT, _OV_BLK), lambda i: (0, i)),
        out_shape=jax.ShapeDtypeStruct((_N_LAT, _N_GENES), jnp.float32),
    )(w_row, lat_col)


def kernel(latent, genes_oi, W_height_slope, W_height_scale, W_height_shift,
           W_overall_slope):
    w_row = W_overall_slope.T
    lat_col = latent[:, None]
    hf = (genes_oi >= _HALF).astype(jnp.int32)[:, None]

    gather = _make_sc_gather()
    gathered = []
    for tab in (W_height_slope, W_height_scale, W_height_shift):
        packed = _repack(tab.T)
        gathered.append(gather(genes_oi, packed))
    g_sl, g_sc, g_sh = gathered

    dh_t = _dh_expand(g_sl, g_sc, g_sh, hf, lat_col)
    ov_t = _ov_expand(w_row, lat_col)

    delta_height = dh_t.transpose(2, 1, 0)
    delta_overall = ov_t.T[:, None, :]
    return (delta_height, delta_overall)


# speedup vs baseline: 2.2576x; 1.0217x over previous
import functools

import jax
import jax.numpy as jnp
from jax import lax
from jax.experimental import pallas as pl
from jax.experimental.pallas import tpu as pltpu
from jax.experimental.pallas import tpu_sc as plsc

_N_GENES = 100000
_N_DH = 64
_N_LAT = 50
_N_GOI = 4096
_HSPL = 50048

_NC, _NS = 2, 16
_NW = _NC * _NS
_B_PER_W = _N_GOI // _NW

_GRID = 16
_BG = _N_GOI // _GRID

_OV_BLK = 8192
_OV_GRID = -(-_N_GENES // _OV_BLK)

_PK_BLK = 2176
_PK_GRID = _HSPL // _PK_BLK


def _pk_body(lo_ref, hi_ref, out_ref):
    out_ref[...] = jnp.concatenate([lo_ref[...].T, hi_ref[...].T], axis=1)


@jax.jit
def _repack(table_t):
    return pl.pallas_call(
        _pk_body,
        grid=(_PK_GRID,),
        in_specs=[
            pl.BlockSpec((_N_DH, _PK_BLK), lambda i: (0, i)),
            pl.BlockSpec((_N_DH, _PK_BLK),
                         lambda i: (0, i + _HSPL // _PK_BLK)),
        ],
        out_specs=pl.BlockSpec((_PK_BLK, 2 * _N_DH), lambda i: (i, 0)),
        out_shape=jax.ShapeDtypeStruct((_HSPL, 2 * _N_DH), jnp.float32),
    )(table_t, table_t)


def _sc_body(idx_hbm, ptab_hbm, out_hbm, idx_v, rows_v, sem):
    wid = lax.axis_index("s") * _NC + lax.axis_index("c")
    base = wid * _B_PER_W
    pltpu.sync_copy(idx_hbm.at[pl.ds(base, _B_PER_W)], idx_v)

    def fix(c, carry):
        v = idx_v[pl.ds(c * 16, 16)]
        idx_v[pl.ds(c * 16, 16)] = v - jnp.where(v >= _HSPL, _HSPL, 0)
        return carry

    lax.fori_loop(0, _B_PER_W // 16, fix, 0)
    pltpu.async_copy(ptab_hbm.at[idx_v], rows_v, sem).wait()
    pltpu.sync_copy(rows_v, out_hbm.at[pl.ds(base, _B_PER_W)])


@functools.cache
def _make_sc_gather():
    return pl.kernel(
        _sc_body,
        out_type=jax.ShapeDtypeStruct((_N_GOI, 2 * _N_DH), jnp.float32),
        mesh=plsc.VectorSubcoreMesh(core_axis_name="c", subcore_axis_name="s",
                                    num_cores=_NC, num_subcores=_NS),
        scratch_types=[
            pltpu.VMEM((_B_PER_W,), jnp.int32),
            pltpu.VMEM((_B_PER_W, 2 * _N_DH), jnp.float32),
            pltpu.SemaphoreType.DMA,
        ],
    )


def _dh_body(sl_ref, sc_ref, sh_ref, hf_ref, lat_ref, out_ref):
    hf = hf_ref[...] > 0

    def sel(ref):
        x = ref[...]
        return jnp.where(hf, x[:, _N_DH:], x[:, : _N_DH])

    h3 = (0.5 * sel(sl_ref)).T[None, :, :]
    a3 = (0.5 * jnp.exp(sel(sc_ref))).T[None, :, :]
    s3 = (0.5 * sel(sh_ref)).T[None, :, :]
    lat3 = lat_ref[...][:, :, None]
    t = jnp.tanh(a3 * lat3 + s3)
    out_ref[...] = h3 * (1.0 + t)


@jax.jit
def _dh_expand(g_sl, g_sc, g_sh, hf, lat_col):
    blk = pl.BlockSpec((_BG, 2 * _N_DH), lambda i: (i, 0))
    return pl.pallas_call(
        _dh_body,
        grid=(_GRID,),
        in_specs=[
            blk, blk, blk,
            pl.BlockSpec((_BG, 1), lambda i: (i, 0)),
            pl.BlockSpec((_N_LAT, 1), lambda i: (0, 0)),
        ],
        out_specs=pl.BlockSpec((_N_LAT, _N_DH, _BG), lambda i: (0, 0, i)),
        out_shape=jax.ShapeDtypeStruct((_N_LAT, _N_DH, _N_GOI), jnp.float32),
    )(g_sl, g_sc, g_sh, hf, lat_col)


def _ov_body(w_ref, lat_ref, out_ref):
    out_ref[...] = lat_ref[...] * w_ref[...]


@jax.jit
def _ov_expand(w_row, lat_col):
    return pl.pallas_call(
        _ov_body,
        grid=(_OV_GRID,),
        in_specs=[
            pl.BlockSpec((1, _OV_BLK), lambda i: (0, i)),
            pl.BlockSpec((_N_LAT, 1), lambda i: (0, 0)),
        ],
        out_specs=pl.BlockSpec((_N_LAT, _OV_BLK), lambda i: (0, i)),
        out_shape=jax.ShapeDtypeStruct((_N_LAT, _N_GENES), jnp.float32),
    )(w_row, lat_col)


def kernel(latent, genes_oi, W_height_slope, W_height_scale, W_height_shift,
           W_overall_slope):
    w_row = W_overall_slope.T
    lat_col = latent[:, None]
    hf = (genes_oi >= _HSPL).astype(jnp.int32)[:, None]

    gather = _make_sc_gather()
    gathered = []
    for tab in (W_height_slope, W_height_scale, W_height_shift):
        packed = _repack(tab.T)
        gathered.append(gather(genes_oi, packed))
    g_sl, g_sc, g_sh = gathered

    dh_t = _dh_expand(g_sl, g_sc, g_sh, hf, lat_col)
    ov_t = _ov_expand(w_row, lat_col)

    delta_height = dh_t.transpose(2, 1, 0)
    delta_overall = ov_t.T[:, None, :]
    return (delta_height, delta_overall)


# speedup vs baseline: 2.7608x; 1.2229x over previous
import functools

import jax
import jax.numpy as jnp
from jax import lax
from jax.experimental import pallas as pl
from jax.experimental.pallas import tpu as pltpu
from jax.experimental.pallas import tpu_sc as plsc

_N_GENES = 100000
_N_DH = 64
_N_LAT = 50
_N_GOI = 4096
_QS = 25088

_NC, _NS = 2, 16
_NW = _NC * _NS
_B_PER_W = _N_GOI // _NW

_GRID = 16
_BG = _N_GOI // _GRID

_OV_BLK = 8192
_OV_GRID = -(-_N_GENES // _OV_BLK)

_PK_BLK = 1792
_PK_GRID = _QS // _PK_BLK


def _pack2(x, y):
    xb = lax.bitcast_convert_type(x.astype(jnp.bfloat16), jnp.uint16)
    yb = lax.bitcast_convert_type(y.astype(jnp.bfloat16), jnp.uint16)
    u = (xb.astype(jnp.uint32) << 16) | yb.astype(jnp.uint32)
    return lax.bitcast_convert_type(u, jnp.float32)


def _pk_body(q0_ref, q1_ref, q2_ref, q3_ref, out_ref):
    out_ref[...] = jnp.concatenate(
        [_pack2(q0_ref[...].T, q1_ref[...].T),
         _pack2(q2_ref[...].T, q3_ref[...].T)], axis=1)


@jax.jit
def _repack(table_t):
    nb = _QS // _PK_BLK
    return pl.pallas_call(
        _pk_body,
        grid=(_PK_GRID,),
        in_specs=[
            pl.BlockSpec((_N_DH, _PK_BLK), lambda i: (0, i)),
            pl.BlockSpec((_N_DH, _PK_BLK), lambda i: (0, i + nb)),
            pl.BlockSpec((_N_DH, _PK_BLK), lambda i: (0, i + 2 * nb)),
            pl.BlockSpec((_N_DH, _PK_BLK), lambda i: (0, i + 3 * nb)),
        ],
        out_specs=pl.BlockSpec((_PK_BLK, 2 * _N_DH), lambda i: (i, 0)),
        out_shape=jax.ShapeDtypeStruct((_QS, 2 * _N_DH), jnp.float32),
    )(table_t, table_t, table_t, table_t)


def _sc_body(idx_hbm, ptab_hbm, out_hbm, idx_v, rows_v, sem):
    wid = lax.axis_index("s") * _NC + lax.axis_index("c")
    base = wid * _B_PER_W
    pltpu.sync_copy(idx_hbm.at[pl.ds(base, _B_PER_W)], idx_v)

    def fix(c, carry):
        v = idx_v[pl.ds(c * 16, 16)]
        idx_v[pl.ds(c * 16, 16)] = v % _QS
        return carry

    lax.fori_loop(0, _B_PER_W // 16, fix, 0)
    pltpu.async_copy(ptab_hbm.at[idx_v], rows_v, sem).wait()
    pltpu.sync_copy(rows_v, out_hbm.at[pl.ds(base, _B_PER_W)])


@functools.cache
def _make_sc_gather():
    return pl.kernel(
        _sc_body,
        out_type=jax.ShapeDtypeStruct((_N_GOI, 2 * _N_DH), jnp.float32),
        mesh=plsc.VectorSubcoreMesh(core_axis_name="c", subcore_axis_name="s",
                                    num_cores=_NC, num_subcores=_NS),
        scratch_types=[
            pltpu.VMEM((_B_PER_W,), jnp.int32),
            pltpu.VMEM((_B_PER_W, 2 * _N_DH), jnp.float32),
            pltpu.SemaphoreType.DMA,
        ],
    )


def _dh_body(sl_ref, sc_ref, sh_ref, qh_ref, ql_ref, lat_ref, out_ref):
    qh = qh_ref[...] > 0
    ql = ql_ref[...] > 0

    def sel(ref):
        x = ref[...]
        u = lax.bitcast_convert_type(
            jnp.where(qh, x[:, _N_DH:], x[:, : _N_DH]), jnp.uint32)
        u = jnp.where(ql, u << 16, u & jnp.uint32(0xFFFF0000))
        return lax.bitcast_convert_type(u, jnp.float32)

    h3 = (0.5 * sel(sl_ref)).T[None, :, :]
    a3 = (0.5 * jnp.exp(sel(sc_ref))).T[None, :, :]
    s3 = (0.5 * sel(sh_ref)).T[None, :, :]
    lat3 = lat_ref[...][:, :, None]
    t = jnp.tanh(a3 * lat3 + s3)
    out_ref[...] = h3 * (1.0 + t)


@jax.jit
def _dh_expand(g_sl, g_sc, g_sh, qh, ql, lat_col):
    blk = pl.BlockSpec((_BG, 2 * _N_DH), lambda i: (i, 0))
    fblk = pl.BlockSpec((_BG, 1), lambda i: (i, 0))
    return pl.pallas_call(
        _dh_body,
        grid=(_GRID,),
        in_specs=[
            blk, blk, blk, fblk, fblk,
            pl.BlockSpec((_N_LAT, 1), lambda i: (0, 0)),
        ],
        out_specs=pl.BlockSpec((_N_LAT, _N_DH, _BG), lambda i: (0, 0, i)),
        out_shape=jax.ShapeDtypeStruct((_N_LAT, _N_DH, _N_GOI), jnp.float32),
    )(g_sl, g_sc, g_sh, qh, ql, lat_col)


def _ov_body(w_ref, lat_ref, out_ref):
    out_ref[...] = lat_ref[...] * w_ref[...]


@jax.jit
def _ov_expand(w_row, lat_col):
    return pl.pallas_call(
        _ov_body,
        grid=(_OV_GRID,),
        in_specs=[
            pl.BlockSpec((1, _OV_BLK), lambda i: (0, i)),
            pl.BlockSpec((_N_LAT, 1), lambda i: (0, 0)),
        ],
        out_specs=pl.BlockSpec((_N_LAT, _OV_BLK), lambda i: (0, i)),
        out_shape=jax.ShapeDtypeStruct((_N_LAT, _N_GENES), jnp.float32),
    )(w_row, lat_col)


def kernel(latent, genes_oi, W_height_slope, W_height_scale, W_height_shift,
           W_overall_slope):
    w_row = W_overall_slope.T
    lat_col = latent[:, None]
    q = genes_oi // _QS
    qh = (q >> 1)[:, None]
    ql = (q & 1)[:, None]

    gather = _make_sc_gather()
    gathered = []
    for tab in (W_height_slope, W_height_scale, W_height_shift):
        packed = _repack(tab.T)
        gathered.append(gather(genes_oi, packed))
    g_sl, g_sc, g_sh = gathered

    dh_t = _dh_expand(g_sl, g_sc, g_sh, qh, ql, lat_col)
    ov_t = _ov_expand(w_row, lat_col)

    delta_height = dh_t.transpose(2, 1, 0)
    delta_overall = ov_t.T[:, None, :]
    return (delta_height, delta_overall)
